# baseline (speedup 1.0000x reference)
import jax
import jax.numpy as jnp
from jax import lax
from jax.experimental import pallas as pl
from jax.experimental.pallas import tpu as pltpu
from jax.experimental.pallas import tpu_sc as plsc

N = 10000
E = 320000
H = 128
G = 64

NC = 2
NS = 16
NW = NC * NS

EROWS = E // 128
TRIPS = 80
EPAD = NW * TRIPS * 128 - E

R = 1000
NBLK = N // R

PROWS = 624
PCHUNK = 104


def _dyn_gather16(vec, idx):
    return lax.gather(
        vec, idx[:, None],
        dimension_numbers=lax.GatherDimensionNumbers(
            offset_dims=(), collapsed_slice_dims=(0,),
            start_index_map=(0,)),
        slice_sizes=(1,),
        mode=lax.GatherScatterMode.PROMISE_IN_BOUNDS)


def _sc_edge_body(h_hbm, as_hbm, ad_hbm, src_hbm, dst_hbm, m_hbm,
                  num_out, den_out,
                  mbuf,
                  sidx0, sidx1, sidx2, sidx3,
                  didx0, didx1, didx2, didx3,
                  esbuf0, esbuf1, edbuf0, edbuf1,
                  eebuf0, eebuf1, rowbuf0, rowbuf1,
                  num_sh, den_sh,
                  gsem0, gsem1, ssem0, ssem1, dsem0, dsem1,
                  isem0, isem1, isem2, isem3):
    c = lax.axis_index("c")
    s = lax.axis_index("s")
    w = c * NS + s

    pltpu.sync_copy(m_hbm, mbuf)
    mv = mbuf[...]

    zero16 = jnp.zeros((16,), jnp.float32)

    def _zero_row(i, carry):
        for k in range(8):
            rowbuf0[i, pl.ds(k * 16, 16)] = zero16
            rowbuf1[i, pl.ds(k * 16, 16)] = zero16
        return carry

    lax.fori_loop(0, 128, _zero_row, 0)
    for j in range(8):
        eebuf0[pl.ds(j * 16, 16)] = zero16
        eebuf1[pl.ds(j * 16, 16)] = zero16

    r0 = s * PROWS
    for t in range(6):
        rr = r0 + t * PCHUNK
        pltpu.sync_copy(rowbuf0.at[pl.ds(0, PCHUNK)],
                        num_sh.at[pl.ds(rr, PCHUNK)])
        pltpu.sync_copy(eebuf0.at[pl.ds(0, PCHUNK)],
                        den_sh.at[pl.ds(rr, PCHUNK)])

    @pl.when(s == NS - 1)
    def _():
        pltpu.sync_copy(rowbuf0.at[pl.ds(0, 16)],
                        num_sh.at[pl.ds(NS * PROWS, 16)])
        pltpu.sync_copy(eebuf0.at[pl.ds(0, 16)],
                        den_sh.at[pl.ds(NS * PROWS, 16)])

    plsc.subcore_barrier()

    bufs = ((rowbuf0, esbuf0, edbuf0, eebuf0, gsem0, ssem0, dsem0),
            (rowbuf1, esbuf1, edbuf1, eebuf1, gsem1, ssem1, dsem1))
    ibufs = ((sidx0, didx0, isem0), (sidx1, didx1, isem1),
             (sidx2, didx2, isem2), (sidx3, didx3, isem3))

    def issue_idx(t, ik):
        si, di, isem = ibufs[ik]
        pltpu.async_copy(src_hbm.at[w, t], si, isem)
        pltpu.async_copy(dst_hbm.at[w, t], di, isem)

    def wait_idx(ik):
        si, di, isem = ibufs[ik]
        pltpu.make_async_copy(src_hbm.at[0, 0], si, isem).wait()
        pltpu.make_async_copy(src_hbm.at[0, 0], di, isem).wait()

    def issue_gather(b, ik):
        rb, es, ed, _, gs, _, _ = bufs[b]
        si, di, _ = ibufs[ik]
        pltpu.async_copy(as_hbm.at[pl.ds(0, 128)], es, gs)
        pltpu.async_copy(ad_hbm.at[pl.ds(0, 128)], ed, gs)
        pltpu.async_copy(h_hbm.at[si], rb, gs)

    def wait_gather_scalars(b):
        rb, es, ed, _, gs, _, _ = bufs[b]
        pltpu.make_async_copy(as_hbm.at[pl.ds(0, 128)], es, gs).wait()
        pltpu.make_async_copy(ad_hbm.at[pl.ds(0, 128)], ed, gs).wait()

    def wait_gather_rows(b):
        rb, _, _, _, gs, _, _ = bufs[b]
        pltpu.make_async_copy(h_hbm.at[pl.ds(0, 128)], rb, gs).wait()

    def issue_scatter_den(b, ik):
        _, _, _, eb, _, _, ds_ = bufs[b]
        _, di, _ = ibufs[ik]
        pltpu.async_copy(eb, den_sh.at[di], ds_, add=True)

    def issue_scatter_rows(b, ik):
        rb, _, _, _, _, ss, _ = bufs[b]
        _, di, _ = ibufs[ik]
        pltpu.async_copy(rb, num_sh.at[di], ss, add=True)

    def wait_scatter(b):
        rb, _, _, eb, _, ss, ds_ = bufs[b]
        pltpu.make_async_copy(rb, num_sh.at[pl.ds(0, 128)], ss).wait()
        pltpu.make_async_copy(eb, den_sh.at[pl.ds(0, 128)], ds_).wait()

    def compute_ee(t, b):
        _, es, ed, eb, _, _, _ = bufs[b]
        valid = (w * TRIPS + t) < EROWS
        vmask = jnp.where(valid, 1.0, 0.0)
        for j in range(8):
            sl = pl.ds(j * 16, 16)
            z = es[sl] + ed[sl]
            e = jnp.where(z > 0, z, 0.2 * z)
            eb[sl] = jnp.exp(e - mv) * vmask

    def scale(b):
        rb, _, _, eb, _, _, _ = bufs[b]

        def _group(j, carry):
            eev = eb[pl.ds(j * 16, 16)]
            for il in range(16):
                i = j * 16 + il
                for k in range(8):
                    sl = pl.ds(k * 16, 16)
                    rb[i, sl] = rb[i, sl] * eev[il]
            return carry

        lax.fori_loop(0, 8, _group, 0)

    issue_idx(0, 0)
    issue_idx(1, 1)
    issue_idx(2, 2)
    wait_idx(0)
    issue_gather(0, 0)
    issue_scatter_den(1, 0)
    issue_scatter_rows(1, 0)

    def half(t, b, ik):
        ob = 1 - b
        ik1 = (ik + 1) % 4
        ik3 = (ik + 3) % 4
        wait_scatter(ob)
        issue_idx(jnp.minimum(t + 3, TRIPS - 1), ik3)
        wait_idx(ik1)
        issue_gather(ob, ik1)
        wait_gather_scalars(b)
        compute_ee(t, b)
        issue_scatter_den(b, ik)
        wait_gather_rows(b)
        scale(b)
        issue_scatter_rows(b, ik)

    def outer(oo, carry):
        t0 = 4 * oo
        half(t0, 0, 0)
        half(t0 + 1, 1, 1)
        half(t0 + 2, 0, 2)
        half(t0 + 3, 1, 3)
        return carry

    lax.fori_loop(0, TRIPS // 4, outer, 0)
    wait_idx(1)
    wait_idx(2)
    wait_gather_scalars(0)
    wait_gather_rows(0)
    wait_scatter(1)
    plsc.subcore_barrier()

    for t in range(6):
        rr = r0 + t * PCHUNK
        pltpu.sync_copy(num_sh.at[pl.ds(rr, PCHUNK)],
                        rowbuf0.at[pl.ds(0, PCHUNK)])
        pltpu.sync_copy(rowbuf0.at[pl.ds(0, PCHUNK)],
                        num_out.at[pl.ds(c * N + rr, PCHUNK)])
        pltpu.sync_copy(den_sh.at[pl.ds(rr, PCHUNK)],
                        eebuf0.at[pl.ds(0, PCHUNK)])
        pltpu.sync_copy(eebuf0.at[pl.ds(0, PCHUNK)],
                        den_out.at[pl.ds(c * N + rr, PCHUNK)])

    @pl.when(s == NS - 1)
    def _():
        rr = NS * PROWS
        pltpu.sync_copy(num_sh.at[pl.ds(rr, 16)], rowbuf0.at[pl.ds(0, 16)])
        pltpu.sync_copy(rowbuf0.at[pl.ds(0, 16)],
                        num_out.at[pl.ds(c * N + rr, 16)])
        pltpu.sync_copy(den_sh.at[pl.ds(rr, 16)], eebuf0.at[pl.ds(0, 16)])
        pltpu.sync_copy(eebuf0.at[pl.ds(0, 16)],
                        den_out.at[pl.ds(c * N + rr, 16)])


@jax.jit
def _sc_edge(h, asv, adv, src2d, dst2d, m16):
    mesh = plsc.VectorSubcoreMesh(core_axis_name="c", subcore_axis_name="s")
    f = pl.kernel(
        _sc_edge_body,
        out_type=(
            jax.ShapeDtypeStruct((NC * N, H), jnp.float32),
            jax.ShapeDtypeStruct((NC * N,), jnp.float32),
        ),
        mesh=mesh,
        scratch_types=(
            [pltpu.VMEM((16,), jnp.float32)]
            + [pltpu.VMEM((128,), jnp.int32) for _ in range(8)]
            + [pltpu.VMEM((128,), jnp.float32) for _ in range(6)]
            + [pltpu.VMEM((128, H), jnp.float32) for _ in range(2)]
            + [pltpu.VMEM_SHARED((N, H), jnp.float32),
               pltpu.VMEM_SHARED((N,), jnp.float32)]
            + [pltpu.SemaphoreType.DMA for _ in range(10)]
        ),
        compiler_params=pltpu.CompilerParams(needs_layout_passes=False),
    )
    num, den = f(h, asv, adv, src2d, dst2d, m16)
    return num.reshape(NC, N, H), den.reshape(NC, N)


def _dense_body(x_ref, w_ref, asv_ref, adv_ref,
                h_ref, as_ref, ad_ref, mxs_ref, mxd_ref):
    h = jnp.dot(x_ref[...], w_ref[...], preferred_element_type=jnp.float32)
    h_ref[...] = h
    a_s = jnp.dot(h, asv_ref[...], preferred_element_type=jnp.float32)
    a_d = jnp.dot(h, adv_ref[...], preferred_element_type=jnp.float32)
    as_ref[...] = a_s
    ad_ref[...] = a_d
    mxs_ref[...] = jnp.full((1, 8, 128), jnp.max(a_s), jnp.float32)
    mxd_ref[...] = jnp.full((1, 8, 128), jnp.max(a_d), jnp.float32)


_DENSE_OUTS = (
    jax.ShapeDtypeStruct((N, H), jnp.float32),
    jax.ShapeDtypeStruct((N, 1), jnp.float32),
    jax.ShapeDtypeStruct((N, 1), jnp.float32),
    jax.ShapeDtypeStruct((NBLK, 8, 128), jnp.float32),
    jax.ShapeDtypeStruct((NBLK, 8, 128), jnp.float32),
)

_DENSE_OUT_SPECS = (
    pl.BlockSpec((R, H), lambda i: (i, 0)),
    pl.BlockSpec((R, 1), lambda i: (i, 0)),
    pl.BlockSpec((R, 1), lambda i: (i, 0)),
    pl.BlockSpec((1, 8, 128), lambda i: (i, 0, 0)),
    pl.BlockSpec((1, 8, 128), lambda i: (i, 0, 0)),
)


@jax.jit
def _dense(x, w, asv, adv):
    return pl.pallas_call(
        _dense_body,
        grid=(NBLK,),
        in_specs=[
            pl.BlockSpec((R, H), lambda i: (i, 0)),
            pl.BlockSpec((H, H), lambda i: (0, 0)),
            pl.BlockSpec((H, 1), lambda i: (0, 0)),
            pl.BlockSpec((H, 1), lambda i: (0, 0)),
        ],
        out_specs=_DENSE_OUT_SPECS,
        out_shape=_DENSE_OUTS,
    )(x, w, asv, adv)


def _finalize_dense_body(num0_ref, num1_ref, den0_ref, den1_ref, b_ref,
                         w_ref, asv_ref, adv_ref,
                         h_ref, as_ref, ad_ref, mxs_ref, mxd_ref):
    dsum = den0_ref[...] + den1_ref[...]
    g = (num0_ref[...] + num1_ref[...]) / (dsum + 1e-16) + b_ref[...]
    hprev = jnp.maximum(g, 0.0)
    h = jnp.dot(hprev, w_ref[...], preferred_element_type=jnp.float32)
    h_ref[...] = h
    a_s = jnp.dot(h, asv_ref[...], preferred_element_type=jnp.float32)
    a_d = jnp.dot(h, adv_ref[...], preferred_element_type=jnp.float32)
    as_ref[...] = a_s
    ad_ref[...] = a_d
    mxs_ref[...] = jnp.full((1, 8, 128), jnp.max(a_s), jnp.float32)
    mxd_ref[...] = jnp.full((1, 8, 128), jnp.max(a_d), jnp.float32)


@jax.jit
def _finalize_dense(num0, num1, den0, den1, b, w, asv, adv):
    return pl.pallas_call(
        _finalize_dense_body,
        grid=(NBLK,),
        in_specs=[
            pl.BlockSpec((R, H), lambda i: (i, 0)),
            pl.BlockSpec((R, H), lambda i: (i, 0)),
            pl.BlockSpec((R, 1), lambda i: (i, 0)),
            pl.BlockSpec((R, 1), lambda i: (i, 0)),
            pl.BlockSpec((1, H), lambda i: (0, 0)),
            pl.BlockSpec((H, H), lambda i: (0, 0)),
            pl.BlockSpec((H, 1), lambda i: (0, 0)),
            pl.BlockSpec((H, 1), lambda i: (0, 0)),
        ],
        out_specs=_DENSE_OUT_SPECS,
        out_shape=_DENSE_OUTS,
    )(num0, num1, den0, den1, b, w, asv, adv)


def _pool_body(num0_ref, num1_ref, den0_ref, den1_ref, b_ref, batch_ref,
               wc_ref, bc_ref, out_ref, pooled, cnt):
    i = pl.program_id(0)

    @pl.when(i == 0)
    def _():
        pooled[...] = jnp.zeros((G, H), jnp.float32)
        cnt[...] = jnp.zeros((G, 128), jnp.float32)

    dsum = den0_ref[...] + den1_ref[...]
    g = (num0_ref[...] + num1_ref[...]) / (dsum + 1e-16) + b_ref[...]
    hf = jnp.maximum(g, 0.0)
    b = batch_ref[...]
    gid = lax.broadcasted_iota(jnp.int32, (R, G), 1)
    onehot = (b == gid).astype(jnp.float32)
    pooled[...] += lax.dot_general(
        onehot, hf, (((0,), (0,)), ((), ())),
        preferred_element_type=jnp.float32)
    cnt[...] += jnp.broadcast_to(
        jnp.sum(onehot, axis=0)[:, None], (G, 128))

    @pl.when(i == NBLK - 1)
    def _():
        pm = pooled[...] / jnp.maximum(cnt[...], 1.0)
        logits = jnp.dot(pm, wc_ref[...],
                         preferred_element_type=jnp.float32) + bc_ref[...]
        out_ref[...] = 1.0 / (1.0 + jnp.exp(-logits))


@jax.jit
def _pool(num0, num1, den0, den1, b, batch2d, wc, bc):
    return pl.pallas_call(
        _pool_body,
        grid=(NBLK,),
        in_specs=[
            pl.BlockSpec((R, H), lambda i: (i, 0)),
            pl.BlockSpec((R, H), lambda i: (i, 0)),
            pl.BlockSpec((R, 1), lambda i: (i, 0)),
            pl.BlockSpec((R, 1), lambda i: (i, 0)),
            pl.BlockSpec((1, H), lambda i: (0, 0)),
            pl.BlockSpec((R, 1), lambda i: (i, 0)),
            pl.BlockSpec((H, 1), lambda i: (0, 0)),
            pl.BlockSpec((1, 1), lambda i: (0, 0)),
        ],
        out_specs=pl.BlockSpec((G, 1), lambda i: (0, 0)),
        out_shape=jax.ShapeDtypeStruct((G, 1), jnp.float32),
        scratch_shapes=[
            pltpu.VMEM((G, H), jnp.float32),
            pltpu.VMEM((G, 128), jnp.float32),
        ],
    )(num0, num1, den0, den1, b, batch2d, wc, bc)


def _lrelu(z):
    return jnp.where(z > 0, z, 0.2 * z)


def kernel(x, edge_index, batch, W1, a_src1, a_dst1, b1,
           W2, a_src2, a_dst2, b2, Wc, bc):
    padidx = (jnp.arange(EPAD, dtype=jnp.int32) * 37) % N
    src2d = jnp.concatenate([edge_index[0], padidx]).reshape(NW, TRIPS, 128)
    dst2d = jnp.concatenate([edge_index[1], padidx]).reshape(NW, TRIPS, 128)

    h1, as1, ad1, mxs1, mxd1 = _dense(
        x, W1, a_src1.reshape(H, 1), a_dst1.reshape(H, 1))
    m1 = _lrelu(jnp.max(mxs1) + jnp.max(mxd1))
    num1, den1 = _sc_edge(h1, as1.reshape(N), ad1.reshape(N),
                          src2d, dst2d,
                          jnp.full((16,), m1, jnp.float32))

    h2, as2, ad2, mxs2, mxd2 = _finalize_dense(
        num1[0], num1[1],
        den1[0].reshape(N, 1), den1[1].reshape(N, 1),
        b1.reshape(1, H), W2, a_src2.reshape(H, 1), a_dst2.reshape(H, 1))
    m2 = _lrelu(jnp.max(mxs2) + jnp.max(mxd2))
    num2, den2 = _sc_edge(h2, as2.reshape(N), ad2.reshape(N),
                          src2d, dst2d,
                          jnp.full((16,), m2, jnp.float32))

    return _pool(num2[0], num2[1],
                 den2[0].reshape(N, 1), den2[1].reshape(N, 1),
                 b2.reshape(1, H), batch.reshape(N, 1), Wc,
                 bc.reshape(1, 1))


# speedup vs baseline: 1.3050x; 1.3050x over previous
import jax
import jax.numpy as jnp
from jax import lax
from jax.experimental import pallas as pl
from jax.experimental.pallas import tpu as pltpu
from jax.experimental.pallas import tpu_sc as plsc

N = 10000
E = 320000
H = 128
G = 64

NC = 2
NS = 16
NW = NC * NS

EROWS = E // 128
TRIPS = 80
EPAD = NW * TRIPS * 128 - E

R = 1000
NBLK = N // R

PROWS = 624
PCHUNK = 104


def _sc_edge_body(h_hbm, as_hbm, ad_hbm, ei_hbm, m_hbm,
                  num_out, den_out,
                  mbuf, ibuf0, ibuf1, ibuf2, ibuf3,
                  eebuf0, eebuf1, bigbuf0, bigbuf1,
                  num_sh, den_sh,
                  gsem0, gsem1, ssem0, ssem1,
                  isem0, isem1, isem2, isem3):
    c = lax.axis_index("c")
    s = lax.axis_index("s")
    w = c * NS + s

    pltpu.sync_copy(m_hbm.at[pl.ds(0, 16)], mbuf)
    mv = mbuf[...]

    zero16 = jnp.zeros((16,), jnp.float32)

    def _zero_row(i, carry):
        for k in range(8):
            bigbuf0[i, pl.ds(k * 16, 16)] = zero16
            bigbuf1[i, pl.ds(k * 16, 16)] = zero16
        return carry

    lax.fori_loop(0, 130, _zero_row, 0)
    for j in range(8):
        eebuf0[pl.ds(j * 16, 16)] = zero16
        eebuf1[pl.ds(j * 16, 16)] = zero16

    r0 = s * PROWS
    for t in range(6):
        rr = r0 + t * PCHUNK
        pltpu.sync_copy(bigbuf0.at[pl.ds(0, PCHUNK)],
                        num_sh.at[pl.ds(rr, PCHUNK)])
        pltpu.sync_copy(eebuf0.at[pl.ds(0, PCHUNK)],
                        den_sh.at[pl.ds(rr, PCHUNK)])

    @pl.when(s == NS - 1)
    def _():
        pltpu.sync_copy(bigbuf0.at[pl.ds(0, 16)],
                        num_sh.at[pl.ds(NS * PROWS, 16)])
        pltpu.sync_copy(eebuf0.at[pl.ds(0, 16)],
                        den_sh.at[pl.ds(NS * PROWS, 16)])

    plsc.subcore_barrier()

    bufs = ((bigbuf0, eebuf0, gsem0, ssem0),
            (bigbuf1, eebuf1, gsem1, ssem1))
    ibufs = ((ibuf0, isem0), (ibuf1, isem1), (ibuf2, isem2),
             (ibuf3, isem3))

    def issue_idx(t, ik):
        ib, isem = ibufs[ik]
        r = jnp.minimum(w * TRIPS + t, EROWS - 1)
        pltpu.async_copy(ei_hbm.at[r], ib, isem)

    def wait_idx(ik):
        ib, isem = ibufs[ik]
        pltpu.make_async_copy(ei_hbm.at[0], ib, isem).wait()

    def issue_gather(b, ik):
        bb, _, gs, _ = bufs[b]
        ib, _ = ibufs[ik]
        pltpu.async_copy(as_hbm.at[ib.at[0]], bb.at[128], gs)
        pltpu.async_copy(ad_hbm.at[ib.at[1]], bb.at[129], gs)
        pltpu.async_copy(h_hbm.at[ib.at[0]], bb.at[pl.ds(0, 128)], gs)

    def wait_gather(b):
        bb, _, gs, _ = bufs[b]
        pltpu.make_async_copy(as_hbm.at[pl.ds(0, 128)], bb.at[128], gs).wait()
        pltpu.make_async_copy(ad_hbm.at[pl.ds(0, 128)], bb.at[129], gs).wait()
        pltpu.make_async_copy(h_hbm.at[pl.ds(0, 128)],
                              bb.at[pl.ds(0, 128)], gs).wait()

    def issue_scatter(b, ik):
        bb, eb, _, ss = bufs[b]
        ib, _ = ibufs[ik]
        pltpu.async_copy(eb, den_sh.at[ib.at[1]], ss, add=True)
        pltpu.async_copy(bb.at[pl.ds(0, 128)], num_sh.at[ib.at[1]], ss,
                         add=True)

    def wait_scatter(b):
        bb, _, _, ss = bufs[b]
        pltpu.make_async_copy(bb.at[pl.ds(0, 129)],
                              num_sh.at[pl.ds(0, 129)], ss).wait()

    def compute_ee(t, b):
        bb, eb, _, _ = bufs[b]
        valid = (w * TRIPS + t) < EROWS
        vmask = jnp.where(valid, 1.0, 0.0)
        for j in range(8):
            sl = pl.ds(j * 16, 16)
            z = bb[128, sl] + bb[129, sl]
            e = jnp.where(z > 0, z, 0.2 * z)
            eb[sl] = jnp.exp(e - mv) * vmask

    def scale(b):
        bb, eb, _, _ = bufs[b]

        def _group(j, carry):
            eev = eb[pl.ds(j * 16, 16)]
            for il in range(16):
                i = j * 16 + il
                for k in range(8):
                    sl = pl.ds(k * 16, 16)
                    bb[i, sl] = bb[i, sl] * eev[il]
            return carry

        lax.fori_loop(0, 8, _group, 0)

    issue_idx(0, 0)
    issue_idx(1, 1)
    issue_idx(2, 2)
    wait_idx(0)
    issue_gather(0, 0)
    issue_scatter(1, 0)

    def half(t, b, ik):
        ob = 1 - b
        ik1 = (ik + 1) % 4
        ik3 = (ik + 3) % 4
        wait_scatter(ob)
        issue_idx(t + 3, ik3)
        wait_idx(ik1)
        issue_gather(ob, ik1)
        wait_gather(b)
        compute_ee(t, b)
        scale(b)
        issue_scatter(b, ik)

    def outer(oo, carry):
        t0 = 4 * oo
        half(t0, 0, 0)
        half(t0 + 1, 1, 1)
        half(t0 + 2, 0, 2)
        half(t0 + 3, 1, 3)
        return carry

    lax.fori_loop(0, TRIPS // 4, outer, 0)
    wait_idx(1)
    wait_idx(2)
    wait_gather(0)
    wait_scatter(1)
    plsc.subcore_barrier()

    for t in range(6):
        rr = r0 + t * PCHUNK
        pltpu.sync_copy(num_sh.at[pl.ds(rr, PCHUNK)],
                        bigbuf0.at[pl.ds(0, PCHUNK)])
        pltpu.sync_copy(bigbuf0.at[pl.ds(0, PCHUNK)],
                        num_out.at[pl.ds(c * N + rr, PCHUNK)])
        pltpu.sync_copy(den_sh.at[pl.ds(rr, PCHUNK)],
                        eebuf0.at[pl.ds(0, PCHUNK)])
        pltpu.sync_copy(eebuf0.at[pl.ds(0, PCHUNK)],
                        den_out.at[pl.ds(c * N + rr, PCHUNK)])

    @pl.when(s == NS - 1)
    def _():
        rr = NS * PROWS
        pltpu.sync_copy(num_sh.at[pl.ds(rr, 16)], bigbuf0.at[pl.ds(0, 16)])
        pltpu.sync_copy(bigbuf0.at[pl.ds(0, 16)],
                        num_out.at[pl.ds(c * N + rr, 16)])
        pltpu.sync_copy(den_sh.at[pl.ds(rr, 16)], eebuf0.at[pl.ds(0, 16)])
        pltpu.sync_copy(eebuf0.at[pl.ds(0, 16)],
                        den_out.at[pl.ds(c * N + rr, 16)])


@jax.jit
def _sc_edge(h, asv, adv, ei3, m16):
    mesh = plsc.VectorSubcoreMesh(core_axis_name="c", subcore_axis_name="s")
    f = pl.kernel(
        _sc_edge_body,
        out_type=(
            jax.ShapeDtypeStruct((NC * N, H), jnp.float32),
            jax.ShapeDtypeStruct((NC * N,), jnp.float32),
        ),
        mesh=mesh,
        scratch_types=(
            [pltpu.VMEM((16,), jnp.float32)]
            + [pltpu.VMEM((2, 128), jnp.int32) for _ in range(4)]
            + [pltpu.VMEM((128,), jnp.float32) for _ in range(2)]
            + [pltpu.VMEM((130, H), jnp.float32) for _ in range(2)]
            + [pltpu.VMEM_SHARED((N, H), jnp.float32),
               pltpu.VMEM_SHARED((N,), jnp.float32)]
            + [pltpu.SemaphoreType.DMA for _ in range(8)]
        ),
        compiler_params=pltpu.CompilerParams(needs_layout_passes=False),
    )
    return f(h, asv, adv, ei3, m16)


def _accum_max_and_offset(i, a_s, a_d, m_ref, mxs_scr, mxd_scr):
    bs = jnp.full((1, 128), jnp.max(a_s), jnp.float32)
    bd = jnp.full((1, 128), jnp.max(a_d), jnp.float32)

    @pl.when(i == 0)
    def _():
        mxs_scr[...] = bs
        mxd_scr[...] = bd

    @pl.when(i > 0)
    def _():
        mxs_scr[...] = jnp.maximum(mxs_scr[...], bs)
        mxd_scr[...] = jnp.maximum(mxd_scr[...], bd)

    @pl.when(i == NBLK - 1)
    def _():
        z = mxs_scr[...] + mxd_scr[...]
        m_ref[...] = jnp.where(z > 0, z, 0.2 * z)


def _dense_body(x_ref, w_ref, asv_ref, adv_ref,
                h_ref, as_ref, ad_ref, m_ref, mxs_scr, mxd_scr):
    h = jnp.dot(x_ref[...], w_ref[...], preferred_element_type=jnp.float32)
    h_ref[...] = h
    a_s = jnp.dot(h, asv_ref[...], preferred_element_type=jnp.float32)
    a_d = jnp.dot(h, adv_ref[...], preferred_element_type=jnp.float32)
    as_ref[...] = a_s
    ad_ref[...] = a_d
    _accum_max_and_offset(pl.program_id(0), a_s, a_d,
                          m_ref, mxs_scr, mxd_scr)


_DENSE_OUTS = (
    jax.ShapeDtypeStruct((N, H), jnp.float32),
    jax.ShapeDtypeStruct((N, 1), jnp.float32),
    jax.ShapeDtypeStruct((N, 1), jnp.float32),
    jax.ShapeDtypeStruct((1, 128), jnp.float32),
)

_DENSE_OUT_SPECS = (
    pl.BlockSpec((R, H), lambda i: (i, 0)),
    pl.BlockSpec((R, 1), lambda i: (i, 0)),
    pl.BlockSpec((R, 1), lambda i: (i, 0)),
    pl.BlockSpec((1, 128), lambda i: (0, 0)),
)

_DENSE_SCRATCH = [
    pltpu.VMEM((1, 128), jnp.float32),
    pltpu.VMEM((1, 128), jnp.float32),
]


@jax.jit
def _dense(x, w, asv, adv):
    return pl.pallas_call(
        _dense_body,
        grid=(NBLK,),
        in_specs=[
            pl.BlockSpec((R, H), lambda i: (i, 0)),
            pl.BlockSpec((H, H), lambda i: (0, 0)),
            pl.BlockSpec((H, 1), lambda i: (0, 0)),
            pl.BlockSpec((H, 1), lambda i: (0, 0)),
        ],
        out_specs=_DENSE_OUT_SPECS,
        out_shape=_DENSE_OUTS,
        scratch_shapes=_DENSE_SCRATCH,
    )(x, w, asv, adv)


def _finalize_dense_body(num0_ref, num1_ref, den0_ref, den1_ref, b_ref,
                         w_ref, asv_ref, adv_ref,
                         h_ref, as_ref, ad_ref, m_ref, mxs_scr, mxd_scr):
    dsum = den0_ref[...] + den1_ref[...]
    g = (num0_ref[...] + num1_ref[...]) / (dsum + 1e-16) + b_ref[...]
    hprev = jnp.maximum(g, 0.0)
    h = jnp.dot(hprev, w_ref[...], preferred_element_type=jnp.float32)
    h_ref[...] = h
    a_s = jnp.dot(h, asv_ref[...], preferred_element_type=jnp.float32)
    a_d = jnp.dot(h, adv_ref[...], preferred_element_type=jnp.float32)
    as_ref[...] = a_s
    ad_ref[...] = a_d
    _accum_max_and_offset(pl.program_id(0), a_s, a_d,
                          m_ref, mxs_scr, mxd_scr)


@jax.jit
def _finalize_dense(num0, num1, den0, den1, b, w, asv, adv):
    return pl.pallas_call(
        _finalize_dense_body,
        grid=(NBLK,),
        in_specs=[
            pl.BlockSpec((R, H), lambda i: (i, 0)),
            pl.BlockSpec((R, H), lambda i: (i + NBLK, 0)),
            pl.BlockSpec((R, 1), lambda i: (i, 0)),
            pl.BlockSpec((R, 1), lambda i: (i + NBLK, 0)),
            pl.BlockSpec((1, H), lambda i: (0, 0)),
            pl.BlockSpec((H, H), lambda i: (0, 0)),
            pl.BlockSpec((H, 1), lambda i: (0, 0)),
            pl.BlockSpec((H, 1), lambda i: (0, 0)),
        ],
        out_specs=_DENSE_OUT_SPECS,
        out_shape=_DENSE_OUTS,
        scratch_shapes=_DENSE_SCRATCH,
    )(num0, num1, den0, den1, b, w, asv, adv)


def _pool_body(num0_ref, num1_ref, den0_ref, den1_ref, b_ref, batch_ref,
               wc_ref, bc_ref, out_ref, pooled, cnt):
    i = pl.program_id(0)

    @pl.when(i == 0)
    def _():
        pooled[...] = jnp.zeros((G, H), jnp.float32)
        cnt[...] = jnp.zeros((G, 128), jnp.float32)

    dsum = den0_ref[...] + den1_ref[...]
    g = (num0_ref[...] + num1_ref[...]) / (dsum + 1e-16) + b_ref[...]
    hf = jnp.maximum(g, 0.0)
    b = batch_ref[...]
    gid = lax.broadcasted_iota(jnp.int32, (R, G), 1)
    onehot = (b == gid).astype(jnp.float32)
    pooled[...] += lax.dot_general(
        onehot, hf, (((0,), (0,)), ((), ())),
        preferred_element_type=jnp.float32)
    cnt[...] += jnp.broadcast_to(
        jnp.sum(onehot, axis=0)[:, None], (G, 128))

    @pl.when(i == NBLK - 1)
    def _():
        pm = pooled[...] / jnp.maximum(cnt[...], 1.0)
        logits = jnp.dot(pm, wc_ref[...],
                         preferred_element_type=jnp.float32) + bc_ref[...]
        out_ref[...] = 1.0 / (1.0 + jnp.exp(-logits))


@jax.jit
def _pool(num0, num1, den0, den1, b, batch2d, wc, bc):
    return pl.pallas_call(
        _pool_body,
        grid=(NBLK,),
        in_specs=[
            pl.BlockSpec((R, H), lambda i: (i, 0)),
            pl.BlockSpec((R, H), lambda i: (i + NBLK, 0)),
            pl.BlockSpec((R, 1), lambda i: (i, 0)),
            pl.BlockSpec((R, 1), lambda i: (i + NBLK, 0)),
            pl.BlockSpec((1, H), lambda i: (0, 0)),
            pl.BlockSpec((R, 1), lambda i: (i, 0)),
            pl.BlockSpec((H, 1), lambda i: (0, 0)),
            pl.BlockSpec((1, 1), lambda i: (0, 0)),
        ],
        out_specs=pl.BlockSpec((G, 1), lambda i: (0, 0)),
        out_shape=jax.ShapeDtypeStruct((G, 1), jnp.float32),
        scratch_shapes=[
            pltpu.VMEM((G, H), jnp.float32),
            pltpu.VMEM((G, 128), jnp.float32),
        ],
    )(num0, num1, den0, den1, b, batch2d, wc, bc)


def kernel(x, edge_index, batch, W1, a_src1, a_dst1, b1,
           W2, a_src2, a_dst2, b2, Wc, bc):
    ei3 = edge_index.reshape(2, EROWS, 128).transpose(1, 0, 2)

    h1, as1, ad1, m1row = _dense(
        x, W1, a_src1.reshape(H, 1), a_dst1.reshape(H, 1))
    num1, den1 = _sc_edge(h1, as1.reshape(N), ad1.reshape(N), ei3,
                          m1row.reshape(128))

    h2, as2, ad2, m2row = _finalize_dense(
        num1, num1, den1.reshape(NC * N, 1), den1.reshape(NC * N, 1),
        b1.reshape(1, H), W2, a_src2.reshape(H, 1), a_dst2.reshape(H, 1))
    num2, den2 = _sc_edge(h2, as2.reshape(N), ad2.reshape(N), ei3,
                          m2row.reshape(128))

    return _pool(num2, num2, den2.reshape(NC * N, 1),
                 den2.reshape(NC * N, 1),
                 b2.reshape(1, H), batch.reshape(N, 1), Wc,
                 bc.reshape(1, 1))
